# Initial kernel scaffold; baseline (speedup 1.0000x reference)
import jax
import jax.numpy as jnp
from jax.experimental import pallas as pl


def kernel(x, W_in, b_in, knowledge_base, W_out, b_out):
    raise NotImplementedError("write your pallas kernel here")


# speedup vs baseline: 22.9415x; 22.9415x over previous
import jax
import jax.numpy as jnp
from jax import lax
from jax.experimental import pallas as pl
from jax.experimental.pallas import tpu as pltpu
from jax.experimental.pallas import tpu_sc as plsc

_M = 64
_NF = 2
_D = 256
_K = 16
_E = 1024
_C = _M * _M

_TILE = 256
_NEG = -1e30

_NW = 32
_TPW = 4096 // _NW
_SUB = 4
_NSUB = _TPW // _SUB


def _log_softmax0(h):
    m = jnp.max(h, axis=0, keepdims=True)
    s = h - m
    return s - jnp.log(jnp.sum(jnp.exp(s), axis=0, keepdims=True))


def _top0_iv(v, k):
    sub = lax.broadcasted_iota(jnp.int32, v.shape, 0)
    vals, idxs = [], []
    for _ in range(k):
        mx = jnp.max(v, axis=0, keepdims=True)
        ii = jnp.min(jnp.where(v == mx, sub, 1 << 30), axis=0, keepdims=True)
        vals.append(mx)
        idxs.append(ii)
        v = jnp.where(sub == ii, _NEG, v)
    return jnp.concatenate(vals, axis=0), jnp.concatenate(idxs, axis=0)


def _topk_body(x_ref, w_in_ref, b_in_ref, idx_ref, wts_ref):
    hT = lax.dot_general(
        w_in_ref[...].astype(jnp.bfloat16), x_ref[...].astype(jnp.bfloat16),
        dimension_numbers=(((0,), (1,)), ((), ())),
        preferred_element_type=jnp.float32)
    hT = hT + b_in_ref[...]
    laT = _log_softmax0(hT[:_M, :])
    lbT = _log_softmax0(hT[_M:, :])

    a16T, iaT = _top0_iv(laT, _K)
    b16T, jbT = _top0_iv(lbT, _K)
    groups = [(0, 16), (1, 8), (2, 5), (3, 4), (4, 3), (5, 2), (6, 2),
              (7, 2)] + [(i, 1) for i in range(8, 16)]
    cparts = [jnp.broadcast_to(a16T[i:i + 1, :], (cnt, _TILE)) + b16T[:cnt, :]
              for i, cnt in groups]
    iparts = [jnp.broadcast_to(iaT[i:i + 1, :] * _M, (cnt, _TILE))
              + jbT[:cnt, :] for i, cnt in groups]
    npad = 64 - sum(cnt for _, cnt in groups)
    cparts.append(jnp.full((npad, _TILE), _NEG, jnp.float32))
    iparts.append(jnp.full((npad, _TILE), 1 << 30, jnp.int32))
    candT = jnp.concatenate(cparts, axis=0)
    cidxT = jnp.concatenate(iparts, axis=0)

    v = candT
    wvals, widxs = [], []
    for _ in range(_K):
        mx = jnp.max(v, axis=0, keepdims=True)
        ki = jnp.min(jnp.where(v == mx, cidxT, 1 << 30), axis=0, keepdims=True)
        wvals.append(jnp.exp(mx))
        widxs.append(ki)
        v = jnp.where((v == mx) & (cidxT == ki), _NEG, v)
    idx_ref[...] = jnp.concatenate(widxs, axis=0)
    wts_ref[...] = jnp.concatenate(wvals, axis=0)


def _topk_call(x2, W_in, b_in):
    grid = x2.shape[0] // _TILE
    return pl.pallas_call(
        _topk_body,
        grid=(grid,),
        in_specs=[
            pl.BlockSpec((_TILE, _E), lambda g: (g, 0)),
            pl.BlockSpec((_E, _NF * _M), lambda g: (0, 0)),
            pl.BlockSpec((_NF * _M, 1), lambda g: (0, 0)),
        ],
        out_specs=[
            pl.BlockSpec((_K, _TILE), lambda g: (0, g)),
            pl.BlockSpec((_K, _TILE), lambda g: (0, g)),
        ],
        out_shape=[
            jax.ShapeDtypeStruct((_K, x2.shape[0]), jnp.int32),
            jax.ShapeDtypeStruct((_K, x2.shape[0]), jnp.float32),
        ],
        compiler_params=pltpu.CompilerParams(
            dimension_semantics=("arbitrary",),
        ),
    )(x2, W_in, b_in.reshape(-1, 1))


def _gather_kernel(kb, idx3, wts2):
    mesh = plsc.VectorSubcoreMesh(core_axis_name="c", subcore_axis_name="s")

    def body(kb_hbm, idx_hbm, wts_hbm, y_hbm, idx_v, wts_v, rows_v, ybuf, sem):
        cid = lax.axis_index("c")
        sid = lax.axis_index("s")
        w = sid * 2 + cid
        pltpu.sync_copy(idx_hbm.at[w], idx_v)
        pltpu.sync_copy(wts_hbm.at[w], wts_v)

        def chunk(j, carry):
            pltpu.async_copy(kb_hbm.at[idx_v.at[j]], rows_v, sem).wait()
            for tt in range(_SUB):
                accs = [jnp.zeros((16,), jnp.float32) for _ in range(_D // 16)]
                wvec = wts_v[pl.ds((j * _SUB + tt) * _K, _K)]
                for k in range(_K):
                    wk = jnp.broadcast_to(wvec[k], (16,))
                    r = tt * _K + k
                    for dc in range(_D // 16):
                        accs[dc] = accs[dc] + wk * rows_v[r, pl.ds(dc * 16, 16)]
                for dc in range(_D // 16):
                    ybuf[tt, pl.ds(dc * 16, 16)] = accs[dc]
            pltpu.sync_copy(ybuf, y_hbm.at[pl.ds(w * _TPW + j * _SUB, _SUB)])
            return carry

        lax.fori_loop(0, _NSUB, chunk, 0)

    f = pl.kernel(
        body,
        mesh=mesh,
        out_type=jax.ShapeDtypeStruct((_NW * _TPW, _D), jnp.float32),
        scratch_types=[
            pltpu.VMEM((_NSUB, _SUB * _K), jnp.int32),
            pltpu.VMEM((_TPW * _K,), jnp.float32),
            pltpu.VMEM((_SUB * _K, _D), jnp.float32),
            pltpu.VMEM((_SUB, _D), jnp.float32),
            pltpu.SemaphoreType.DMA,
        ],
    )
    return f(kb, idx3, wts2)


def _out_body(y_ref, w_out_ref, b_out_ref, out_ref):
    out = lax.dot_general(
        y_ref[...].astype(jnp.bfloat16), w_out_ref[...].astype(jnp.bfloat16),
        dimension_numbers=(((1,), (0,)), ((), ())),
        preferred_element_type=jnp.float32)
    out_ref[...] = out + b_out_ref[...]


def _out_call(y, W_out, b_out):
    grid = y.shape[0] // 1024
    return pl.pallas_call(
        _out_body,
        grid=(grid,),
        in_specs=[
            pl.BlockSpec((1024, _D), lambda g: (g, 0)),
            pl.BlockSpec((_D, _E), lambda g: (0, 0)),
            pl.BlockSpec((1, _E), lambda g: (0, 0)),
        ],
        out_specs=pl.BlockSpec((1024, _E), lambda g: (g, 0)),
        out_shape=jax.ShapeDtypeStruct((y.shape[0], _E), jnp.float32),
        compiler_params=pltpu.CompilerParams(
            dimension_semantics=("arbitrary",),
        ),
    )(y, W_out, b_out.reshape(1, -1))


@jax.jit
def kernel(x, W_in, b_in, knowledge_base, W_out, b_out):
    B, T, E = x.shape
    tokens = B * T
    x2 = x.reshape(tokens, E)

    idxT, wtsT = _topk_call(x2, W_in, b_in)
    idx3 = idxT.T.reshape(_NW, _NSUB, _SUB * _K)
    wts2 = wtsT.T.reshape(_NW, _TPW * _K)
    y = _gather_kernel(knowledge_base, idx3, wts2)
    out = _out_call(y, W_out, b_out)
    return out.reshape(B, T, E)
